# baseline (speedup 1.0000x reference)
import functools

import jax
import jax.numpy as jnp
import numpy as np
from jax.experimental import pallas as pl
from jax.experimental.pallas import tpu as pltpu

B = 1
S = 2048
D = 768
H = 12
HD = 64
HH = HD // 2
E = 8
TOPK = 2
FFN = 768
BASE = 10000.0
EPS = 1e-5

ST = 256
QT = 512
NEG = jnp.finfo(jnp.float32).min
BF = jnp.bfloat16
F32 = jnp.float32


def _ln(x, scale):
    mu = jnp.mean(x, axis=-1, keepdims=True)
    var = jnp.mean(jnp.square(x - mu), axis=-1, keepdims=True)
    return (x - mu) / jnp.sqrt(var + EPS) * scale


def _qkv_body(x_ref, s1_ref, wq_ref, wk_ref, wv_ref, cos_ref, sin_ref,
              q_ref, k_ref, v_ref):
    h = _ln(x_ref[...], s1_ref[...]).astype(BF)
    cos = cos_ref[...]
    sin = sin_ref[...]

    q = jnp.dot(h, wq_ref[...], preferred_element_type=F32)
    q1 = q[:, : H * HH]
    q2 = q[:, H * HH:]
    scale = 1.0 / np.sqrt(HD)
    q_ref[:, : H * HH] = ((q1 * cos - q2 * sin) * scale).astype(BF)
    q_ref[:, H * HH:] = ((q2 * cos + q1 * sin) * scale).astype(BF)

    k = jnp.dot(h, wk_ref[...], preferred_element_type=F32)
    k1 = k[:, : H * HH]
    k2 = k[:, H * HH:]
    k_ref[:, : H * HH] = (k1 * cos - k2 * sin).astype(BF)
    k_ref[:, H * HH:] = (k2 * cos + k1 * sin).astype(BF)

    v_ref[...] = jnp.dot(h, wv_ref[...], preferred_element_type=F32).astype(BF)


def _attn_body(q_ref, k_ref, v_ref, o_ref, acc_ref, m_ref, l_ref):
    qi = pl.program_id(0)
    ki = pl.program_id(1)

    @pl.when(ki <= qi)
    def _():
        qpos = qi * QT + jax.lax.broadcasted_iota(jnp.int32, (QT, QT), 0)
        kpos = ki * QT + jax.lax.broadcasted_iota(jnp.int32, (QT, QT), 1)
        causal = kpos <= qpos
        first = ki == 0
        last = ki == qi
        for h in range(H):
            q1 = q_ref[:, HH * h: HH * h + HH]
            q2 = q_ref[:, H * HH + HH * h: H * HH + HH * h + HH]
            k1 = k_ref[:, HH * h: HH * h + HH]
            k2 = k_ref[:, H * HH + HH * h: H * HH + HH * h + HH]
            dn = (((1,), (1,)), ((), ()))
            s = jax.lax.dot_general(q1, k1, dn, preferred_element_type=F32)
            s = s + jax.lax.dot_general(q2, k2, dn, preferred_element_type=F32)
            s = jnp.where(causal, s, NEG)
            m_cur = jnp.max(s, axis=1, keepdims=True)
            m_old = jnp.where(first, NEG, m_ref[:, h: h + 1])
            m_new = jnp.maximum(m_old, m_cur)
            corr = jnp.exp(m_old - m_new)
            p = jnp.exp(s - m_new)
            ps = jnp.sum(p, axis=1, keepdims=True)
            l_new = jnp.where(first, ps, l_ref[:, h: h + 1] * corr + ps)
            vh = v_ref[:, HD * h: HD * h + HD]
            pv = jnp.dot(p.astype(BF), vh, preferred_element_type=F32)
            acc = jnp.where(first, pv, acc_ref[:, HD * h: HD * h + HD] * corr + pv)
            m_ref[:, h: h + 1] = m_new
            l_ref[:, h: h + 1] = l_new
            acc_ref[:, HD * h: HD * h + HD] = acc

            @pl.when(last)
            def _():
                o_ref[:, HD * h: HD * h + HD] = (acc / l_new).astype(BF)


def _router_body(a_ref, wo_ref, x_ref, s2_ref, rw_ref,
                 r2_ref, h2_ref, w_ref, g_ref):
    attn = jnp.dot(a_ref[...], wo_ref[...], preferred_element_type=F32)
    resid2 = x_ref[...] + attn
    r2_ref[...] = resid2
    h2 = _ln(resid2, s2_ref[...])
    h2_ref[...] = h2.astype(BF)
    logits = jnp.dot(h2, rw_ref[...], preferred_element_type=F32)
    m = jnp.max(logits, axis=1, keepdims=True)
    ew = jnp.exp(logits - m)
    w = ew / jnp.sum(ew, axis=1, keepdims=True)
    w_ref[...] = w
    eidx = jax.lax.broadcasted_iota(jnp.int32, (ST, E), 1)
    e1 = jnp.argmax(w, axis=1)[:, None]
    oh1 = eidx == e1
    m1 = jnp.max(w, axis=1, keepdims=True)
    masked = jnp.where(oh1, -1.0, w)
    e2 = jnp.argmax(masked, axis=1)[:, None]
    oh2 = eidx == e2
    m2 = jnp.max(masked, axis=1, keepdims=True)
    denom = m1 + m2
    g_ref[...] = (jnp.where(oh1, m1 / denom, 0.0)
                  + jnp.where(oh2, m2 / denom, 0.0))


def _moe_body(h2_ref, r2_ref, g_ref, w1_ref, v1_ref, w2_ref, o_ref):
    e = pl.program_id(0)
    eidx = jax.lax.broadcasted_iota(jnp.int32, (S, E), 1)
    gate = jnp.sum(jnp.where(eidx == e, g_ref[...], 0.0), axis=1, keepdims=True)
    h2 = h2_ref[...]
    dn_t = (((1,), (1,)), ((), ()))
    x1 = jax.lax.dot_general(h2, w1_ref[...], dn_t, preferred_element_type=F32)
    x2 = jax.lax.dot_general(h2, v1_ref[...], dn_t, preferred_element_type=F32)
    gl = (x1 * jax.nn.sigmoid(x1) * x2).astype(BF)
    y = jnp.dot(gl, w2_ref[...], preferred_element_type=F32)
    contrib = gate * y

    @pl.when(e == 0)
    def _():
        o_ref[...] = r2_ref[...] + contrib

    @pl.when(e != 0)
    def _():
        o_ref[...] = o_ref[...] + contrib


def _build_tables():
    inv_freq = 1.0 / (BASE ** (np.arange(0, HD, 2, dtype=np.float32) / HD))
    pos = np.arange(S, dtype=np.float32)
    freqs = pos[:, None] * inv_freq[None, :]
    cos = np.tile(np.cos(freqs), (1, H)).astype(np.float32)
    sin = np.tile(np.sin(freqs), (1, H)).astype(np.float32)
    perm = np.concatenate([
        np.concatenate([np.arange(HH) + HD * h for h in range(H)]),
        np.concatenate([np.arange(HH) + HH + HD * h for h in range(H)]),
    ])
    return cos, sin, perm


_COS, _SIN, _PERM = _build_tables()


@jax.jit
def kernel(x, ln1_scale, ln2_scale, Wqkv, Wout, router_w, w1, v1, w2):
    xf = x.reshape(S, D)
    wq = Wqkv[:, :D][:, _PERM].astype(BF)
    wk = Wqkv[:, D:2 * D][:, _PERM].astype(BF)
    wv = Wqkv[:, 2 * D:].astype(BF)
    wo = Wout.astype(BF)
    w1b = w1.astype(BF)
    v1b = v1.astype(BF)
    w2b = w2.astype(BF)
    s1 = ln1_scale.reshape(1, D)
    s2 = ln2_scale.reshape(1, D)

    nst = S // ST

    q, k, v = pl.pallas_call(
        _qkv_body,
        grid=(nst,),
        in_specs=[
            pl.BlockSpec((ST, D), lambda i: (i, 0)),
            pl.BlockSpec((1, D), lambda i: (0, 0)),
            pl.BlockSpec((D, D), lambda i: (0, 0)),
            pl.BlockSpec((D, D), lambda i: (0, 0)),
            pl.BlockSpec((D, D), lambda i: (0, 0)),
            pl.BlockSpec((ST, H * HH), lambda i: (i, 0)),
            pl.BlockSpec((ST, H * HH), lambda i: (i, 0)),
        ],
        out_specs=[
            pl.BlockSpec((ST, D), lambda i: (i, 0)),
            pl.BlockSpec((ST, D), lambda i: (i, 0)),
            pl.BlockSpec((ST, D), lambda i: (i, 0)),
        ],
        out_shape=[jax.ShapeDtypeStruct((S, D), BF)] * 3,
    )(xf, s1, wq, wk, wv, _COS, _SIN)

    attn = pl.pallas_call(
        _attn_body,
        grid=(S // QT, S // QT),
        in_specs=[
            pl.BlockSpec((QT, D), lambda i, j: (i, 0)),
            pl.BlockSpec((QT, D), lambda i, j: (j, 0)),
            pl.BlockSpec((QT, D), lambda i, j: (j, 0)),
        ],
        out_specs=pl.BlockSpec((QT, D), lambda i, j: (i, 0)),
        out_shape=jax.ShapeDtypeStruct((S, D), BF),
        scratch_shapes=[
            pltpu.VMEM((QT, D), F32),
            pltpu.VMEM((QT, 128), F32),
            pltpu.VMEM((QT, 128), F32),
        ],
    )(q, k, v)

    resid2, h2, weights, gates = pl.pallas_call(
        _router_body,
        grid=(nst,),
        in_specs=[
            pl.BlockSpec((ST, D), lambda i: (i, 0)),
            pl.BlockSpec((D, D), lambda i: (0, 0)),
            pl.BlockSpec((ST, D), lambda i: (i, 0)),
            pl.BlockSpec((1, D), lambda i: (0, 0)),
            pl.BlockSpec((D, E), lambda i: (0, 0)),
        ],
        out_specs=[
            pl.BlockSpec((ST, D), lambda i: (i, 0)),
            pl.BlockSpec((ST, D), lambda i: (i, 0)),
            pl.BlockSpec((ST, E), lambda i: (i, 0)),
            pl.BlockSpec((ST, E), lambda i: (i, 0)),
        ],
        out_shape=[
            jax.ShapeDtypeStruct((S, D), F32),
            jax.ShapeDtypeStruct((S, D), BF),
            jax.ShapeDtypeStruct((S, E), F32),
            jax.ShapeDtypeStruct((S, E), F32),
        ],
    )(attn, wo, xf, s2, router_w)

    out = pl.pallas_call(
        _moe_body,
        grid=(E,),
        in_specs=[
            pl.BlockSpec((S, D), lambda e: (0, 0)),
            pl.BlockSpec((S, D), lambda e: (0, 0)),
            pl.BlockSpec((S, E), lambda e: (0, 0)),
            pl.BlockSpec((FFN, D), lambda e: (e, 0)),
            pl.BlockSpec((FFN, D), lambda e: (e, 0)),
            pl.BlockSpec((FFN, D), lambda e: (e, 0)),
        ],
        out_specs=pl.BlockSpec((S, D), lambda e: (0, 0)),
        out_shape=jax.ShapeDtypeStruct((S, D), F32),
    )(h2, resid2, gates, w1b, v1b, w2b)

    return out.reshape(B, S, D), weights.reshape(B, S, E)


# speedup vs baseline: 1.2783x; 1.2783x over previous
import functools

import jax
import jax.numpy as jnp
import numpy as np
from jax.experimental import pallas as pl
from jax.experimental.pallas import tpu as pltpu

B = 1
S = 2048
D = 768
H = 12
HD = 64
HH = HD // 2
E = 8
TOPK = 2
FFN = 768
BASE = 10000.0
EPS = 1e-5

ST = 256
QT = 512
NEG = jnp.finfo(jnp.float32).min
BF = jnp.bfloat16
F32 = jnp.float32


def _ln(x, scale):
    mu = jnp.mean(x, axis=-1, keepdims=True)
    var = jnp.mean(jnp.square(x - mu), axis=-1, keepdims=True)
    return (x - mu) / jnp.sqrt(var + EPS) * scale


def _qkv_body(x_ref, s1_ref, wq_ref, wk_ref, wv_ref, cos_ref, sin_ref,
              q_ref, k_ref, v_ref):
    h = _ln(x_ref[...], s1_ref[...]).astype(BF)
    cos = cos_ref[...]
    sin = sin_ref[...]

    q = jnp.dot(h, wq_ref[...], preferred_element_type=F32)
    q1 = q[:, : H * HH]
    q2 = q[:, H * HH:]
    scale = 1.0 / np.sqrt(HD)
    q_ref[:, : H * HH] = ((q1 * cos - q2 * sin) * scale).astype(BF)
    q_ref[:, H * HH:] = ((q2 * cos + q1 * sin) * scale).astype(BF)

    k = jnp.dot(h, wk_ref[...], preferred_element_type=F32)
    k1 = k[:, : H * HH]
    k2 = k[:, H * HH:]
    k_ref[:, : H * HH] = (k1 * cos - k2 * sin).astype(BF)
    k_ref[:, H * HH:] = (k2 * cos + k1 * sin).astype(BF)

    v_ref[...] = jnp.dot(h, wv_ref[...], preferred_element_type=F32).astype(BF)


def _attn_body(q_ref, k_ref, v_ref, o_ref):
    i = pl.program_id(0)
    qpos = i * QT + jax.lax.broadcasted_iota(jnp.int32, (QT, S), 0)
    kpos = jax.lax.broadcasted_iota(jnp.int32, (QT, S), 1)
    causal = kpos <= qpos
    for h in range(H):
        q1 = q_ref[:, HH * h: HH * h + HH]
        q2 = q_ref[:, H * HH + HH * h: H * HH + HH * h + HH]
        k1 = k_ref[:, HH * h: HH * h + HH]
        k2 = k_ref[:, H * HH + HH * h: H * HH + HH * h + HH]
        dn = (((1,), (1,)), ((), ()))
        s = jax.lax.dot_general(q1, k1, dn, preferred_element_type=F32)
        s = s + jax.lax.dot_general(q2, k2, dn, preferred_element_type=F32)
        s = jnp.where(causal, s, NEG)
        m = jnp.max(s, axis=1, keepdims=True)
        p = jnp.exp(s - m)
        l = jnp.sum(p, axis=1, keepdims=True)
        vh = v_ref[:, HD * h: HD * h + HD]
        pv = jnp.dot(p.astype(BF), vh, preferred_element_type=F32)
        o_ref[:, HD * h: HD * h + HD] = (pv / l).astype(BF)


def _router_body(a_ref, wo_ref, x_ref, s2_ref, rw_ref,
                 r2_ref, h2_ref, w_ref, g_ref):
    attn = jnp.dot(a_ref[...], wo_ref[...], preferred_element_type=F32)
    resid2 = x_ref[...] + attn
    r2_ref[...] = resid2
    h2 = _ln(resid2, s2_ref[...])
    h2_ref[...] = h2.astype(BF)
    logits = jnp.dot(h2, rw_ref[...], preferred_element_type=F32)
    m = jnp.max(logits, axis=1, keepdims=True)
    ew = jnp.exp(logits - m)
    w = ew / jnp.sum(ew, axis=1, keepdims=True)
    w_ref[...] = w
    eidx = jax.lax.broadcasted_iota(jnp.int32, (ST, E), 1)
    e1 = jnp.argmax(w, axis=1)[:, None]
    oh1 = eidx == e1
    m1 = jnp.max(w, axis=1, keepdims=True)
    masked = jnp.where(oh1, -1.0, w)
    e2 = jnp.argmax(masked, axis=1)[:, None]
    oh2 = eidx == e2
    m2 = jnp.max(masked, axis=1, keepdims=True)
    denom = m1 + m2
    g_ref[...] = (jnp.where(oh1, m1 / denom, 0.0)
                  + jnp.where(oh2, m2 / denom, 0.0))


def _moe_body(h2_ref, r2_ref, g_ref, w1_ref, v1_ref, w2_ref, o_ref):
    e = pl.program_id(0)
    eidx = jax.lax.broadcasted_iota(jnp.int32, (S, E), 1)
    gate = jnp.sum(jnp.where(eidx == e, g_ref[...], 0.0), axis=1, keepdims=True)
    h2 = h2_ref[...]
    dn_t = (((1,), (1,)), ((), ()))
    x1 = jax.lax.dot_general(h2, w1_ref[...], dn_t, preferred_element_type=F32)
    x2 = jax.lax.dot_general(h2, v1_ref[...], dn_t, preferred_element_type=F32)
    gl = (x1 * jax.nn.sigmoid(x1) * x2).astype(BF)
    y = jnp.dot(gl, w2_ref[...], preferred_element_type=F32)
    contrib = gate * y

    @pl.when(e == 0)
    def _():
        o_ref[...] = r2_ref[...] + contrib

    @pl.when(e != 0)
    def _():
        o_ref[...] = o_ref[...] + contrib


def _build_tables():
    inv_freq = 1.0 / (BASE ** (np.arange(0, HD, 2, dtype=np.float32) / HD))
    pos = np.arange(S, dtype=np.float32)
    freqs = pos[:, None] * inv_freq[None, :]
    cos = np.tile(np.cos(freqs), (1, H)).astype(np.float32)
    sin = np.tile(np.sin(freqs), (1, H)).astype(np.float32)
    perm = np.concatenate([
        np.concatenate([np.arange(HH) + HD * h for h in range(H)]),
        np.concatenate([np.arange(HH) + HH + HD * h for h in range(H)]),
    ])
    return cos, sin, perm


_COS, _SIN, _PERM = _build_tables()


@jax.jit
def kernel(x, ln1_scale, ln2_scale, Wqkv, Wout, router_w, w1, v1, w2):
    xf = x.reshape(S, D)
    wq = Wqkv[:, :D][:, _PERM].astype(BF)
    wk = Wqkv[:, D:2 * D][:, _PERM].astype(BF)
    wv = Wqkv[:, 2 * D:].astype(BF)
    wo = Wout.astype(BF)
    w1b = w1.astype(BF)
    v1b = v1.astype(BF)
    w2b = w2.astype(BF)
    s1 = ln1_scale.reshape(1, D)
    s2 = ln2_scale.reshape(1, D)

    nst = S // ST

    q, k, v = pl.pallas_call(
        _qkv_body,
        grid=(nst,),
        in_specs=[
            pl.BlockSpec((ST, D), lambda i: (i, 0)),
            pl.BlockSpec((1, D), lambda i: (0, 0)),
            pl.BlockSpec((D, D), lambda i: (0, 0)),
            pl.BlockSpec((D, D), lambda i: (0, 0)),
            pl.BlockSpec((D, D), lambda i: (0, 0)),
            pl.BlockSpec((ST, H * HH), lambda i: (i, 0)),
            pl.BlockSpec((ST, H * HH), lambda i: (i, 0)),
        ],
        out_specs=[
            pl.BlockSpec((ST, D), lambda i: (i, 0)),
            pl.BlockSpec((ST, D), lambda i: (i, 0)),
            pl.BlockSpec((ST, D), lambda i: (i, 0)),
        ],
        out_shape=[jax.ShapeDtypeStruct((S, D), BF)] * 3,
    )(xf, s1, wq, wk, wv, _COS, _SIN)

    attn = pl.pallas_call(
        _attn_body,
        grid=(S // QT,),
        in_specs=[
            pl.BlockSpec((QT, D), lambda i: (i, 0)),
            pl.BlockSpec((S, D), lambda i: (0, 0)),
            pl.BlockSpec((S, D), lambda i: (0, 0)),
        ],
        out_specs=pl.BlockSpec((QT, D), lambda i: (i, 0)),
        out_shape=jax.ShapeDtypeStruct((S, D), BF),
    )(q, k, v)

    resid2, h2, weights, gates = pl.pallas_call(
        _router_body,
        grid=(nst,),
        in_specs=[
            pl.BlockSpec((ST, D), lambda i: (i, 0)),
            pl.BlockSpec((D, D), lambda i: (0, 0)),
            pl.BlockSpec((ST, D), lambda i: (i, 0)),
            pl.BlockSpec((1, D), lambda i: (0, 0)),
            pl.BlockSpec((D, E), lambda i: (0, 0)),
        ],
        out_specs=[
            pl.BlockSpec((ST, D), lambda i: (i, 0)),
            pl.BlockSpec((ST, D), lambda i: (i, 0)),
            pl.BlockSpec((ST, E), lambda i: (i, 0)),
            pl.BlockSpec((ST, E), lambda i: (i, 0)),
        ],
        out_shape=[
            jax.ShapeDtypeStruct((S, D), F32),
            jax.ShapeDtypeStruct((S, D), BF),
            jax.ShapeDtypeStruct((S, E), F32),
            jax.ShapeDtypeStruct((S, E), F32),
        ],
    )(attn, wo, xf, s2, router_w)

    out = pl.pallas_call(
        _moe_body,
        grid=(E,),
        in_specs=[
            pl.BlockSpec((S, D), lambda e: (0, 0)),
            pl.BlockSpec((S, D), lambda e: (0, 0)),
            pl.BlockSpec((S, E), lambda e: (0, 0)),
            pl.BlockSpec((FFN, D), lambda e: (e, 0)),
            pl.BlockSpec((FFN, D), lambda e: (e, 0)),
            pl.BlockSpec((FFN, D), lambda e: (e, 0)),
        ],
        out_specs=pl.BlockSpec((S, D), lambda e: (0, 0)),
        out_shape=jax.ShapeDtypeStruct((S, D), F32),
    )(h2, resid2, gates, w1b, v1b, w2b)

    return out.reshape(B, S, D), weights.reshape(B, S, E)
